# baseline (speedup 1.0000x reference)
import functools

import jax
import jax.numpy as jnp
from jax import lax
from jax.experimental import pallas as pl
from jax.experimental.pallas import tpu as pltpu
from jax.experimental.pallas import tpu_sc as plsc

N = 10000
E = 320000
C = 128
CH = C // 2
NG = 128

NC = 2
NS = 16
NW = NC * NS
EPW = E // NW
EPS = E // NS
CHUNK = 125
NCHUNK = EPS // CHUNK
SUB_ROWS = 624
TAIL_OFF = NS * SUB_ROWS
TAIL_ROWS = N - TAIL_OFF

DEPTH = 5
LOOK = 2


def _degree_body(dst_hbm, out_hbm, dst_v, deg_v):
    cid = lax.axis_index("c")
    sid = lax.axis_index("s")
    wid = cid * NS + sid
    pltpu.sync_copy(dst_hbm.at[wid], dst_v)

    def zero_body(i, _):
        deg_v[pl.ds(i * 16, 16)] = jnp.zeros((16,), jnp.float32)
        return 0

    lax.fori_loop(0, N // 16, zero_body, 0)

    ones = jnp.ones((16,), jnp.float32)

    def add_body(i, _):
        idx = dst_v[i, :]
        plsc.addupdate_scatter(deg_v, [idx], ones)
        return 0

    lax.fori_loop(0, EPW // 16, add_body, 0)
    pltpu.sync_copy(deg_v, out_hbm.at[wid])


def _conv_body(ts_hbm, zeros_hbm, src_hbm, dst_hbm, out_hbm,
               src_v, dst_v, *rest):
    bufs = rest[:DEPTH]
    acc_sh = rest[DEPTH]
    gsem = rest[DEPTH + 1:2 * DEPTH + 1]
    ssem = rest[2 * DEPTH + 1:]
    cid = lax.axis_index("c")
    sid = lax.axis_index("s")
    pltpu.sync_copy(src_hbm.at[cid, sid], src_v)
    pltpu.sync_copy(dst_hbm.at[sid], dst_v)

    base = sid * SUB_ROWS
    pltpu.sync_copy(zeros_hbm.at[pl.ds(base, SUB_ROWS)],
                    acc_sh.at[pl.ds(base, SUB_ROWS)])

    @pl.when(sid == NS - 1)
    def _():
        pltpu.sync_copy(zeros_hbm.at[pl.ds(TAIL_OFF, TAIL_ROWS)],
                        acc_sh.at[pl.ds(TAIL_OFF, TAIL_ROWS)])

    plsc.subcore_barrier()

    def gstart(k, j):
        pltpu.async_copy(ts_hbm.at[src_v.at[k]], bufs[j], gsem[j])

    def gdrain(j):
        pltpu.make_async_copy(ts_hbm.at[src_v.at[0]], bufs[j], gsem[j]).wait()

    def sstart(k, j):
        pltpu.async_copy(bufs[j], acc_sh.at[dst_v.at[k]], ssem[j], add=True)

    def sdrain(j):
        pltpu.make_async_copy(bufs[j], acc_sh.at[dst_v.at[0]], ssem[j]).wait()

    for j in range(DEPTH):
        gstart(j, j)
    for j in range(DEPTH - LOOK):
        gdrain(j)
        sstart(j, j)

    def chunk_body(i, _):
        for j in range(DEPTH):
            k = DEPTH * i + j
            sdrain(j)
            gstart(k, j)
            jj = (j + DEPTH - LOOK) % DEPTH
            gdrain(jj)
            sstart(k - LOOK, jj)
        return 0

    lax.fori_loop(1, NCHUNK // DEPTH, chunk_body, 0)
    for t in range(LOOK):
        jj = DEPTH - LOOK + t
        gdrain(jj)
        sstart(NCHUNK - LOOK + t, jj)
    for j in range(DEPTH):
        sdrain(j)
    plsc.subcore_barrier()

    pltpu.sync_copy(acc_sh.at[pl.ds(base, SUB_ROWS)],
                    out_hbm.at[pl.ds(base, SUB_ROWS), cid])

    @pl.when(sid == NS - 1)
    def _():
        pltpu.sync_copy(acc_sh.at[pl.ds(TAIL_OFF, TAIL_ROWS)],
                        out_hbm.at[pl.ds(TAIL_OFF, TAIL_ROWS), cid])


@functools.cache
def _sc_kernels():
    mesh = plsc.VectorSubcoreMesh(core_axis_name="c", subcore_axis_name="s")
    params = pltpu.CompilerParams(needs_layout_passes=False,
                                  use_tc_tiling_on_sc=False)
    sc_degree = functools.partial(
        pl.kernel,
        out_type=jax.ShapeDtypeStruct((NW, N), jnp.float32),
        mesh=mesh,
        compiler_params=params,
        scratch_types=[
            pltpu.VMEM((EPW // 16, 16), jnp.int32),
            pltpu.VMEM((N,), jnp.float32),
        ],
    )(_degree_body)
    sc_conv = functools.partial(
        pl.kernel,
        out_type=jax.ShapeDtypeStruct((N, NC, CH), jnp.float32),
        mesh=mesh,
        compiler_params=params,
        scratch_types=[
            pltpu.VMEM((NCHUNK, CHUNK), jnp.int32),
            pltpu.VMEM((NCHUNK, CHUNK), jnp.int32),
        ] + [pltpu.VMEM((CHUNK, CH), jnp.float32)] * DEPTH
          + [
            pltpu.VMEM_SHARED((N, CH), jnp.float32),
        ] + [pltpu.SemaphoreType.DMA] * (2 * DEPTH),
    )(_conv_body)
    return sc_degree, sc_conv


def _tc1_body(x_ref, w1_ref, part_ref, xw1s_ref, dinv_ref):
    ones = jnp.ones((NW, C), jnp.float32)
    dn = (((0,), (0,)), ((), ()))
    deg = lax.dot_general(part_ref[...], ones, dn,
                          preferred_element_type=jnp.float32) + 1.0
    dinv = lax.rsqrt(deg)
    xw = jnp.dot(x_ref[...], w1_ref[...], preferred_element_type=jnp.float32)
    xw1s_ref[...] = xw * dinv
    dinv_ref[...] = dinv


_tc1 = pl.pallas_call(
    _tc1_body,
    out_shape=(jax.ShapeDtypeStruct((N, C), jnp.float32),
               jax.ShapeDtypeStruct((N, C), jnp.float32)),
)


def _bn_relu(s, g, be):
    mu = jnp.mean(s, axis=0, keepdims=True)
    xc = s - mu
    var = jnp.mean(xc * xc, axis=0, keepdims=True)
    h = xc * lax.rsqrt(var + 1e-5) * g + be
    return jnp.maximum(h, 0.0)


def _tc2_body(p_ref, ts_ref, dinv_ref, b_ref, g_ref, be_ref, w_ref, out_ref):
    dinv = dinv_ref[...]
    s = (p_ref[...] + ts_ref[...]) * dinv + b_ref[...]
    h = _bn_relu(s, g_ref[...], be_ref[...])
    out_ref[...] = jnp.dot(h, w_ref[...],
                           preferred_element_type=jnp.float32) * dinv


_tc2 = pl.pallas_call(
    _tc2_body,
    out_shape=jax.ShapeDtypeStruct((N, C), jnp.float32),
)


def _tc3_body(p_ref, ts_ref, dinv_ref, b_ref, g_ref, be_ref, batch_ref,
              wl_ref, bl_ref, out_ref):
    s = (p_ref[...] + ts_ref[...]) * dinv_ref[...] + b_ref[...]
    h = _bn_relu(s, g_ref[...], be_ref[...])
    gid = lax.broadcasted_iota(jnp.int32, (N, NG), 1)
    m = (batch_ref[...] == gid).astype(jnp.float32)
    cnt = jnp.sum(m, axis=0, keepdims=True)
    m = m * (1.0 / jnp.maximum(cnt, 1.0))
    dn = (((0,), (0,)), ((), ()))
    pooled = lax.dot_general(m, h, dn, preferred_element_type=jnp.float32)
    out_ref[...] = jnp.dot(pooled, wl_ref[...],
                           preferred_element_type=jnp.float32) + bl_ref[...]


_tc3 = pl.pallas_call(
    _tc3_body,
    out_shape=jax.ShapeDtypeStruct((NG, C), jnp.float32),
)


def kernel(x, edge_index, batch, W1, b1, g1, be1, W2, b2, g2, be2, Wl, bl):
    src = edge_index[0].astype(jnp.int32)
    dst = edge_index[1].astype(jnp.int32)
    off = jnp.arange(NC, dtype=jnp.int32)[:, None]
    src4 = (src[None, :] * 2 + off).reshape(NC, NS, NCHUNK, CHUNK)
    dst3 = dst.reshape(NS, NCHUNK, CHUNK)
    dst16 = dst.reshape(NW, EPW // 16, 16)
    batch_b = jnp.broadcast_to(batch.astype(jnp.int32)[:, None], (N, NG))

    zeros = jnp.zeros((N, CH), jnp.float32)
    sc_degree, sc_conv = _sc_kernels()
    deg_part = sc_degree(dst16)
    xw1s, dinv = _tc1(x, W1, deg_part)
    p1 = sc_conv(xw1s.reshape(NC * N, CH), zeros, src4, dst3)
    xw2s = _tc2(p1.reshape(N, C), xw1s, dinv, b1, g1, be1, W2)
    p2 = sc_conv(xw2s.reshape(NC * N, CH), zeros, src4, dst3)
    return _tc3(p2.reshape(N, C), xw2s, dinv, b2, g2, be2, batch_b, Wl, bl)


# speedup vs baseline: 1.2860x; 1.2860x over previous
import functools

import jax
import jax.numpy as jnp
from jax import lax
from jax.experimental import pallas as pl
from jax.experimental.pallas import tpu as pltpu
from jax.experimental.pallas import tpu_sc as plsc

N = 10000
E = 320000
C = 128
CH = C // 2
NG = 128

NC = 2
NS = 16
NW = NC * NS
EPW = E // NW
EPS = E // NS
CHUNK = 125
NCHUNK = EPS // CHUNK
SUB_ROWS = 624
TAIL_OFF = NS * SUB_ROWS
TAIL_ROWS = N - TAIL_OFF

DEPTH = 5
LOOK = 2


def _degree_body(dst_hbm, out_hbm, dst_v, deg_v):
    cid = lax.axis_index("c")
    sid = lax.axis_index("s")
    wid = cid * NS + sid
    pltpu.sync_copy(dst_hbm.at[wid], dst_v)

    def zero_body(i, _):
        deg_v[pl.ds(i * 16, 16)] = jnp.zeros((16,), jnp.float32)
        return 0

    lax.fori_loop(0, N // 16, zero_body, 0)

    ones = jnp.ones((16,), jnp.float32)

    def add_body(i, _):
        idx = dst_v[i, :]
        plsc.addupdate_scatter(deg_v, [idx], ones)
        return 0

    lax.fori_loop(0, EPW // 16, add_body, 0)
    pltpu.sync_copy(deg_v, out_hbm.at[wid])


def _conv_body(ts_hbm, zeros_hbm, src_hbm, dst_hbm, out_hbm,
               src_v, dst_v, *rest):
    bufs = rest[:DEPTH]
    acc_sh = rest[DEPTH]
    gsem = rest[DEPTH + 1:2 * DEPTH + 1]
    ssem = rest[2 * DEPTH + 1:]
    cid = lax.axis_index("c")
    sid = lax.axis_index("s")
    pltpu.sync_copy(src_hbm.at[cid, sid], src_v)
    pltpu.sync_copy(dst_hbm.at[sid], dst_v)

    base = sid * SUB_ROWS
    pltpu.sync_copy(zeros_hbm.at[pl.ds(base, SUB_ROWS)],
                    acc_sh.at[pl.ds(base, SUB_ROWS)])

    @pl.when(sid == NS - 1)
    def _():
        pltpu.sync_copy(zeros_hbm.at[pl.ds(TAIL_OFF, TAIL_ROWS)],
                        acc_sh.at[pl.ds(TAIL_OFF, TAIL_ROWS)])

    plsc.subcore_barrier()

    def gstart(k, j):
        pltpu.async_copy(ts_hbm.at[src_v.at[k]], bufs[j], gsem[j])

    def gdrain(j):
        pltpu.make_async_copy(ts_hbm.at[src_v.at[0]], bufs[j], gsem[j]).wait()

    def sstart(k, j):
        pltpu.async_copy(bufs[j], acc_sh.at[dst_v.at[k]], ssem[j], add=True)

    def sdrain(j):
        pltpu.make_async_copy(bufs[j], acc_sh.at[dst_v.at[0]], ssem[j]).wait()

    for j in range(DEPTH):
        gstart(j, j)
    for j in range(DEPTH - LOOK):
        gdrain(j)
        sstart(j, j)

    def chunk_body(i, _):
        for j in range(DEPTH):
            k = DEPTH * i + j
            sdrain(j)
            gstart(k, j)
            jj = (j + DEPTH - LOOK) % DEPTH
            gdrain(jj)
            sstart(k - LOOK, jj)
        return 0

    lax.fori_loop(1, NCHUNK // DEPTH, chunk_body, 0)
    for t in range(LOOK):
        jj = DEPTH - LOOK + t
        gdrain(jj)
        sstart(NCHUNK - LOOK + t, jj)
    for j in range(DEPTH):
        sdrain(j)
    plsc.subcore_barrier()

    col = cid * CH
    pltpu.sync_copy(acc_sh.at[pl.ds(base, SUB_ROWS)],
                    out_hbm.at[pl.ds(base, SUB_ROWS), pl.ds(col, CH)])

    @pl.when(sid == NS - 1)
    def _():
        pltpu.sync_copy(acc_sh.at[pl.ds(TAIL_OFF, TAIL_ROWS)],
                        out_hbm.at[pl.ds(TAIL_OFF, TAIL_ROWS), pl.ds(col, CH)])


@functools.cache
def _sc_kernels():
    mesh = plsc.VectorSubcoreMesh(core_axis_name="c", subcore_axis_name="s")
    params = pltpu.CompilerParams(needs_layout_passes=False,
                                  use_tc_tiling_on_sc=False)
    sc_degree = functools.partial(
        pl.kernel,
        out_type=jax.ShapeDtypeStruct((NW, N), jnp.float32),
        mesh=mesh,
        compiler_params=params,
        scratch_types=[
            pltpu.VMEM((EPW // 16, 16), jnp.int32),
            pltpu.VMEM((N,), jnp.float32),
        ],
    )(_degree_body)
    sc_conv = functools.partial(
        pl.kernel,
        out_type=jax.ShapeDtypeStruct((N, C), jnp.float32),
        mesh=mesh,
        compiler_params=params,
        scratch_types=[
            pltpu.VMEM((NCHUNK, CHUNK), jnp.int32),
            pltpu.VMEM((NCHUNK, CHUNK), jnp.int32),
        ] + [pltpu.VMEM((CHUNK, CH), jnp.float32)] * DEPTH
          + [
            pltpu.VMEM_SHARED((N, CH), jnp.float32),
        ] + [pltpu.SemaphoreType.DMA] * (2 * DEPTH),
    )(_conv_body)
    return sc_degree, sc_conv


def _tc1_body(x_ref, w1_ref, part_ref, xw1s_ref, dinv_ref):
    ones = jnp.ones((NW, C), jnp.float32)
    dn = (((0,), (0,)), ((), ()))
    deg = lax.dot_general(part_ref[...], ones, dn,
                          preferred_element_type=jnp.float32) + 1.0
    dinv = lax.rsqrt(deg)
    xw = jnp.dot(x_ref[...], w1_ref[...], preferred_element_type=jnp.float32)
    xw1s_ref[...] = xw * dinv
    dinv_ref[...] = dinv


_tc1 = pl.pallas_call(
    _tc1_body,
    out_shape=(jax.ShapeDtypeStruct((N, C), jnp.float32),
               jax.ShapeDtypeStruct((N, C), jnp.float32)),
)


def _bn_relu(s, g, be):
    mu = jnp.mean(s, axis=0, keepdims=True)
    xc = s - mu
    var = jnp.mean(xc * xc, axis=0, keepdims=True)
    h = xc * lax.rsqrt(var + 1e-5) * g + be
    return jnp.maximum(h, 0.0)


def _tc2_body(p_ref, ts_ref, dinv_ref, b_ref, g_ref, be_ref, w_ref, out_ref):
    dinv = dinv_ref[...]
    s = (p_ref[...] + ts_ref[...]) * dinv + b_ref[...]
    h = _bn_relu(s, g_ref[...], be_ref[...])
    out_ref[...] = jnp.dot(h, w_ref[...],
                           preferred_element_type=jnp.float32) * dinv


_tc2 = pl.pallas_call(
    _tc2_body,
    out_shape=jax.ShapeDtypeStruct((N, C), jnp.float32),
)


def _tc3_body(p_ref, ts_ref, dinv_ref, b_ref, g_ref, be_ref, batch_ref,
              wl_ref, bl_ref, out_ref):
    s = (p_ref[...] + ts_ref[...]) * dinv_ref[...] + b_ref[...]
    h = _bn_relu(s, g_ref[...], be_ref[...])
    gid = lax.broadcasted_iota(jnp.int32, (N, NG), 1)
    m = (batch_ref[...] == gid).astype(jnp.float32)
    cnt = jnp.sum(m, axis=0, keepdims=True)
    m = m * (1.0 / jnp.maximum(cnt, 1.0))
    dn = (((0,), (0,)), ((), ()))
    pooled = lax.dot_general(m, h, dn, preferred_element_type=jnp.float32)
    out_ref[...] = jnp.dot(pooled, wl_ref[...],
                           preferred_element_type=jnp.float32) + bl_ref[...]


_tc3 = pl.pallas_call(
    _tc3_body,
    out_shape=jax.ShapeDtypeStruct((NG, C), jnp.float32),
)


def kernel(x, edge_index, batch, W1, b1, g1, be1, W2, b2, g2, be2, Wl, bl):
    src = edge_index[0].astype(jnp.int32)
    dst = edge_index[1].astype(jnp.int32)
    off = jnp.arange(NC, dtype=jnp.int32)[:, None]
    src4 = (src[None, :] * 2 + off).reshape(NC, NS, NCHUNK, CHUNK)
    dst3 = dst.reshape(NS, NCHUNK, CHUNK)
    dst16 = dst.reshape(NW, EPW // 16, 16)
    batch_b = jnp.broadcast_to(batch.astype(jnp.int32)[:, None], (N, NG))

    zeros = jnp.zeros((N, CH), jnp.float32)
    sc_degree, sc_conv = _sc_kernels()
    deg_part = sc_degree(dst16)
    xw1s, dinv = _tc1(x, W1, deg_part)
    p1 = sc_conv(xw1s.reshape(NC * N, CH), zeros, src4, dst3)
    xw2s = _tc2(p1, xw1s, dinv, b1, g1, be1, W2)
    p2 = sc_conv(xw2s.reshape(NC * N, CH), zeros, src4, dst3)
    return _tc3(p2, xw2s, dinv, b2, g2, be2, batch_b, Wl, bl)


# speedup vs baseline: 1.3124x; 1.0205x over previous
import functools

import jax
import jax.numpy as jnp
from jax import lax
from jax.experimental import pallas as pl
from jax.experimental.pallas import tpu as pltpu
from jax.experimental.pallas import tpu_sc as plsc

N = 10000
E = 320000
C = 128
CH = C // 2
NG = 128

NC = 2
NS = 16
NW = NC * NS
EPW = E // NW
EPS = E // NS
CHUNK = 125
NCHUNK = EPS // CHUNK
SUB_ROWS = 624
TAIL_OFF = NS * SUB_ROWS
TAIL_ROWS = N - TAIL_OFF

DEPTH = 5
LOOK = 2


def _degree_body(ei_hbm, out_hbm, dst_v, deg_v):
    cid = lax.axis_index("c")
    sid = lax.axis_index("s")
    wid = cid * NS + sid
    pltpu.sync_copy(ei_hbm.at[1, pl.ds(wid * EPW, EPW)], dst_v)

    def zero_body(i, _):
        deg_v[pl.ds(i * 16, 16)] = jnp.zeros((16,), jnp.float32)
        return 0

    lax.fori_loop(0, N // 16, zero_body, 0)

    ones = jnp.ones((16,), jnp.float32)

    def add_body(i, _):
        idx = dst_v[pl.ds(i * 16, 16)]
        plsc.addupdate_scatter(deg_v, [idx], ones)
        return 0

    lax.fori_loop(0, EPW // 16, add_body, 0)
    pltpu.sync_copy(deg_v, out_hbm.at[wid])


def _conv_body(ts_hbm, zeros_hbm, src_hbm, dst_hbm, out_hbm,
               src_v, dst_v, *rest):
    bufs = rest[:DEPTH]
    acc_sh = rest[DEPTH]
    gsem = rest[DEPTH + 1:2 * DEPTH + 1]
    ssem = rest[2 * DEPTH + 1:]
    cid = lax.axis_index("c")
    sid = lax.axis_index("s")
    pltpu.sync_copy(src_hbm.at[cid, sid], src_v)
    pltpu.sync_copy(dst_hbm.at[sid], dst_v)

    base = sid * SUB_ROWS
    pltpu.sync_copy(zeros_hbm.at[pl.ds(base, SUB_ROWS)],
                    acc_sh.at[pl.ds(base, SUB_ROWS)])

    @pl.when(sid == NS - 1)
    def _():
        pltpu.sync_copy(zeros_hbm.at[pl.ds(TAIL_OFF, TAIL_ROWS)],
                        acc_sh.at[pl.ds(TAIL_OFF, TAIL_ROWS)])

    plsc.subcore_barrier()

    def gstart(k, j):
        pltpu.async_copy(ts_hbm.at[src_v.at[k]], bufs[j], gsem[j])

    def gdrain(j):
        pltpu.make_async_copy(ts_hbm.at[src_v.at[0]], bufs[j], gsem[j]).wait()

    def sstart(k, j):
        pltpu.async_copy(bufs[j], acc_sh.at[dst_v.at[k]], ssem[j], add=True)

    def sdrain(j):
        pltpu.make_async_copy(bufs[j], acc_sh.at[dst_v.at[0]], ssem[j]).wait()

    for j in range(DEPTH):
        gstart(j, j)
    for j in range(DEPTH - LOOK):
        gdrain(j)
        sstart(j, j)

    def chunk_body(i, _):
        for j in range(DEPTH):
            k = DEPTH * i + j
            sdrain(j)
            gstart(k, j)
            jj = (j + DEPTH - LOOK) % DEPTH
            gdrain(jj)
            sstart(k - LOOK, jj)
        return 0

    lax.fori_loop(1, NCHUNK // DEPTH, chunk_body, 0)
    for t in range(LOOK):
        jj = DEPTH - LOOK + t
        gdrain(jj)
        sstart(NCHUNK - LOOK + t, jj)
    for j in range(DEPTH):
        sdrain(j)
    plsc.subcore_barrier()

    col = cid * CH
    pltpu.sync_copy(acc_sh.at[pl.ds(base, SUB_ROWS)],
                    out_hbm.at[pl.ds(base, SUB_ROWS), pl.ds(col, CH)])

    @pl.when(sid == NS - 1)
    def _():
        pltpu.sync_copy(acc_sh.at[pl.ds(TAIL_OFF, TAIL_ROWS)],
                        out_hbm.at[pl.ds(TAIL_OFF, TAIL_ROWS), pl.ds(col, CH)])


@functools.cache
def _sc_kernels():
    mesh = plsc.VectorSubcoreMesh(core_axis_name="c", subcore_axis_name="s")
    params = pltpu.CompilerParams(needs_layout_passes=False,
                                  use_tc_tiling_on_sc=False)
    sc_degree = functools.partial(
        pl.kernel,
        out_type=jax.ShapeDtypeStruct((NW, N), jnp.float32),
        mesh=mesh,
        compiler_params=params,
        scratch_types=[
            pltpu.VMEM((EPW,), jnp.int32),
            pltpu.VMEM((N,), jnp.float32),
        ],
    )(_degree_body)
    sc_conv = functools.partial(
        pl.kernel,
        out_type=jax.ShapeDtypeStruct((N, C), jnp.float32),
        mesh=mesh,
        compiler_params=params,
        scratch_types=[
            pltpu.VMEM((NCHUNK, CHUNK), jnp.int32),
            pltpu.VMEM((NCHUNK, CHUNK), jnp.int32),
        ] + [pltpu.VMEM((CHUNK, CH), jnp.float32)] * DEPTH
          + [
            pltpu.VMEM_SHARED((N, CH), jnp.float32),
        ] + [pltpu.SemaphoreType.DMA] * (2 * DEPTH),
    )(_conv_body)
    return sc_degree, sc_conv


def _tc1_body(x_ref, w1_ref, part_ref, xw1s_ref, dinv_ref):
    ones = jnp.ones((NW, C), jnp.float32)
    dn = (((0,), (0,)), ((), ()))
    deg = lax.dot_general(part_ref[...], ones, dn,
                          preferred_element_type=jnp.float32) + 1.0
    dinv = lax.rsqrt(deg)
    xw = jnp.dot(x_ref[...], w1_ref[...], preferred_element_type=jnp.float32)
    xw1s_ref[...] = xw * dinv
    dinv_ref[...] = dinv


_tc1 = pl.pallas_call(
    _tc1_body,
    out_shape=(jax.ShapeDtypeStruct((N, C), jnp.float32),
               jax.ShapeDtypeStruct((N, C), jnp.float32)),
)


def _bn_relu(s, g, be):
    mu = jnp.mean(s, axis=0, keepdims=True)
    xc = s - mu
    var = jnp.mean(xc * xc, axis=0, keepdims=True)
    h = xc * lax.rsqrt(var + 1e-5) * g + be
    return jnp.maximum(h, 0.0)


def _tc2_body(p_ref, ts_ref, dinv_ref, b_ref, g_ref, be_ref, w_ref, out_ref):
    dinv = dinv_ref[...]
    s = (p_ref[...] + ts_ref[...]) * dinv + b_ref[...]
    h = _bn_relu(s, g_ref[...], be_ref[...])
    out_ref[...] = jnp.dot(h, w_ref[...],
                           preferred_element_type=jnp.float32) * dinv


_tc2 = pl.pallas_call(
    _tc2_body,
    out_shape=jax.ShapeDtypeStruct((N, C), jnp.float32),
)


def _tc3_body(p_ref, ts_ref, dinv_ref, b_ref, g_ref, be_ref, lo_ref, hi_ref,
              wl_ref, bl_ref, out_ref):
    s = (p_ref[...] + ts_ref[...]) * dinv_ref[...] + b_ref[...]
    h = _bn_relu(s, g_ref[...], be_ref[...])
    lo, hi = lo_ref[...], hi_ref[...]
    ii = lax.broadcasted_iota(jnp.int32, (N, NG), 0)
    m = ((ii >= lo) & (ii < hi)).astype(jnp.float32)
    cnt = (hi - lo).astype(jnp.float32)
    m = m * (1.0 / jnp.maximum(cnt, 1.0))
    dn = (((0,), (0,)), ((), ()))
    pooled = lax.dot_general(m, h, dn, preferred_element_type=jnp.float32)
    out_ref[...] = jnp.dot(pooled, wl_ref[...],
                           preferred_element_type=jnp.float32) + bl_ref[...]


_tc3 = pl.pallas_call(
    _tc3_body,
    out_shape=jax.ShapeDtypeStruct((NG, C), jnp.float32),
)


def kernel(x, edge_index, batch, W1, b1, g1, be1, W2, b2, g2, be2, Wl, bl):
    src = edge_index[0].astype(jnp.int32)
    dst = edge_index[1].astype(jnp.int32)
    off = jnp.arange(NC, dtype=jnp.int32)[:, None]
    src4 = (src[None, :] * 2 + off).reshape(NC, NS, NCHUNK, CHUNK)
    dst3 = dst.reshape(NS, NCHUNK, CHUNK)
    b32 = batch.astype(jnp.int32)
    g129 = jnp.arange(NG + 1, dtype=jnp.int32)
    lohi = jnp.sum((b32[:, None] < g129[None, :]).astype(jnp.int32), axis=0)
    lo = lohi[:NG].reshape(1, NG)
    hi = lohi[1:].reshape(1, NG)

    zeros = jnp.zeros((N, CH), jnp.float32)
    sc_degree, sc_conv = _sc_kernels()
    ei32 = edge_index.astype(jnp.int32)
    deg_part = sc_degree(ei32)
    xw1s, dinv = _tc1(x, W1, deg_part)
    p1 = sc_conv(xw1s.reshape(NC * N, CH), zeros, src4, dst3)
    xw2s = _tc2(p1, xw1s, dinv, b1, g1, be1, W2)
    p2 = sc_conv(xw2s.reshape(NC * N, CH), zeros, src4, dst3)
    return _tc3(p2, xw2s, dinv, b2, g2, be2, lo, hi, Wl, bl)


# speedup vs baseline: 1.3459x; 1.0255x over previous
import functools

import jax
import jax.numpy as jnp
from jax import lax
from jax.experimental import pallas as pl
from jax.experimental.pallas import tpu as pltpu
from jax.experimental.pallas import tpu_sc as plsc

N = 10000
E = 320000
C = 128
CH = C // 2
NG = 128

NC = 2
NS = 16
NW = NC * NS
EPW = E // NW
EPS = E // NS
CHUNK = 125
NCHUNK = EPS // CHUNK
SUB_ROWS = 624
TAIL_OFF = NS * SUB_ROWS
TAIL_ROWS = N - TAIL_OFF

DEPTH = 5
LOOK = 3


def _degree_body(ei_hbm, out_hbm, dst_v, deg_v):
    cid = lax.axis_index("c")
    sid = lax.axis_index("s")
    wid = cid * NS + sid
    pltpu.sync_copy(ei_hbm.at[1, pl.ds(wid * EPW, EPW)], dst_v)

    def zero_body(i, _):
        deg_v[pl.ds(i * 16, 16)] = jnp.zeros((16,), jnp.float32)
        return 0

    lax.fori_loop(0, N // 16, zero_body, 0)

    ones = jnp.ones((16,), jnp.float32)

    def add_body(i, _):
        idx = dst_v[pl.ds(i * 16, 16)]
        plsc.addupdate_scatter(deg_v, [idx], ones)
        return 0

    lax.fori_loop(0, EPW // 16, add_body, 0)
    pltpu.sync_copy(deg_v, out_hbm.at[wid])


def _conv_body(ts_hbm, zeros_hbm, src_hbm, dst_hbm, out_hbm,
               src_v, dst_v, *rest):
    bufs = rest[:DEPTH]
    acc_sh = rest[DEPTH]
    gsem = rest[DEPTH + 1:2 * DEPTH + 1]
    ssem = rest[2 * DEPTH + 1:]
    cid = lax.axis_index("c")
    sid = lax.axis_index("s")
    pltpu.sync_copy(src_hbm.at[cid, sid], src_v)
    pltpu.sync_copy(dst_hbm.at[sid], dst_v)

    base = sid * SUB_ROWS
    pltpu.sync_copy(zeros_hbm.at[pl.ds(base, SUB_ROWS)],
                    acc_sh.at[pl.ds(base, SUB_ROWS)])

    @pl.when(sid == NS - 1)
    def _():
        pltpu.sync_copy(zeros_hbm.at[pl.ds(TAIL_OFF, TAIL_ROWS)],
                        acc_sh.at[pl.ds(TAIL_OFF, TAIL_ROWS)])

    plsc.subcore_barrier()

    def gstart(k, j):
        pltpu.async_copy(ts_hbm.at[src_v.at[k]], bufs[j], gsem[j])

    def gdrain(j):
        pltpu.make_async_copy(ts_hbm.at[src_v.at[0]], bufs[j], gsem[j]).wait()

    def sstart(k, j):
        pltpu.async_copy(bufs[j], acc_sh.at[dst_v.at[k]], ssem[j], add=True)

    def sdrain(j):
        pltpu.make_async_copy(bufs[j], acc_sh.at[dst_v.at[0]], ssem[j]).wait()

    for j in range(DEPTH):
        gstart(j, j)
    for j in range(DEPTH - LOOK):
        gdrain(j)
        sstart(j, j)

    def chunk_body(i, _):
        for j in range(DEPTH):
            k = DEPTH * i + j
            sdrain(j)
            gstart(k, j)
            jj = (j + DEPTH - LOOK) % DEPTH
            gdrain(jj)
            sstart(k - LOOK, jj)
        return 0

    lax.fori_loop(1, NCHUNK // DEPTH, chunk_body, 0)
    for t in range(LOOK):
        jj = DEPTH - LOOK + t
        gdrain(jj)
        sstart(NCHUNK - LOOK + t, jj)
    for j in range(DEPTH):
        sdrain(j)
    plsc.subcore_barrier()

    col = cid * CH
    pltpu.sync_copy(acc_sh.at[pl.ds(base, SUB_ROWS)],
                    out_hbm.at[pl.ds(base, SUB_ROWS), pl.ds(col, CH)])

    @pl.when(sid == NS - 1)
    def _():
        pltpu.sync_copy(acc_sh.at[pl.ds(TAIL_OFF, TAIL_ROWS)],
                        out_hbm.at[pl.ds(TAIL_OFF, TAIL_ROWS), pl.ds(col, CH)])


@functools.cache
def _sc_kernels():
    mesh = plsc.VectorSubcoreMesh(core_axis_name="c", subcore_axis_name="s")
    params = pltpu.CompilerParams(needs_layout_passes=False,
                                  use_tc_tiling_on_sc=False)
    sc_degree = functools.partial(
        pl.kernel,
        out_type=jax.ShapeDtypeStruct((NW, N), jnp.float32),
        mesh=mesh,
        compiler_params=params,
        scratch_types=[
            pltpu.VMEM((EPW,), jnp.int32),
            pltpu.VMEM((N,), jnp.float32),
        ],
    )(_degree_body)
    sc_conv = functools.partial(
        pl.kernel,
        out_type=jax.ShapeDtypeStruct((N, C), jnp.float32),
        mesh=mesh,
        compiler_params=params,
        scratch_types=[
            pltpu.VMEM((NCHUNK, CHUNK), jnp.int32),
            pltpu.VMEM((NCHUNK, CHUNK), jnp.int32),
        ] + [pltpu.VMEM((CHUNK, CH), jnp.float32)] * DEPTH
          + [
            pltpu.VMEM_SHARED((N, CH), jnp.float32),
        ] + [pltpu.SemaphoreType.DMA] * (2 * DEPTH),
    )(_conv_body)
    return sc_degree, sc_conv


def _tc1_body(x_ref, w1_ref, part_ref, xw1s_ref, dinv_ref):
    ones = jnp.ones((NW, C), jnp.float32)
    dn = (((0,), (0,)), ((), ()))
    deg = lax.dot_general(part_ref[...], ones, dn,
                          preferred_element_type=jnp.float32) + 1.0
    dinv = lax.rsqrt(deg)
    xw = jnp.dot(x_ref[...], w1_ref[...], preferred_element_type=jnp.float32)
    xw1s_ref[...] = xw * dinv
    dinv_ref[...] = dinv


_tc1 = pl.pallas_call(
    _tc1_body,
    out_shape=(jax.ShapeDtypeStruct((N, C), jnp.float32),
               jax.ShapeDtypeStruct((N, C), jnp.float32)),
)


def _bn_relu(s, g, be):
    mu = jnp.mean(s, axis=0, keepdims=True)
    xc = s - mu
    var = jnp.mean(xc * xc, axis=0, keepdims=True)
    h = xc * lax.rsqrt(var + 1e-5) * g + be
    return jnp.maximum(h, 0.0)


def _tc2_body(p_ref, ts_ref, dinv_ref, b_ref, g_ref, be_ref, w_ref, out_ref):
    dinv = dinv_ref[...]
    s = (p_ref[...] + ts_ref[...]) * dinv + b_ref[...]
    h = _bn_relu(s, g_ref[...], be_ref[...])
    out_ref[...] = jnp.dot(h, w_ref[...],
                           preferred_element_type=jnp.float32) * dinv


_tc2 = pl.pallas_call(
    _tc2_body,
    out_shape=jax.ShapeDtypeStruct((N, C), jnp.float32),
)


def _tc3_body(p_ref, ts_ref, dinv_ref, b_ref, g_ref, be_ref, lo_ref, hi_ref,
              wl_ref, bl_ref, out_ref):
    s = (p_ref[...] + ts_ref[...]) * dinv_ref[...] + b_ref[...]
    h = _bn_relu(s, g_ref[...], be_ref[...])
    lo, hi = lo_ref[...], hi_ref[...]
    ii = lax.broadcasted_iota(jnp.int32, (N, NG), 0)
    m = ((ii >= lo) & (ii < hi)).astype(jnp.float32)
    cnt = (hi - lo).astype(jnp.float32)
    m = m * (1.0 / jnp.maximum(cnt, 1.0))
    dn = (((0,), (0,)), ((), ()))
    pooled = lax.dot_general(m, h, dn, preferred_element_type=jnp.float32)
    out_ref[...] = jnp.dot(pooled, wl_ref[...],
                           preferred_element_type=jnp.float32) + bl_ref[...]


_tc3 = pl.pallas_call(
    _tc3_body,
    out_shape=jax.ShapeDtypeStruct((NG, C), jnp.float32),
)


def kernel(x, edge_index, batch, W1, b1, g1, be1, W2, b2, g2, be2, Wl, bl):
    src = edge_index[0].astype(jnp.int32)
    dst = edge_index[1].astype(jnp.int32)
    off = jnp.arange(NC, dtype=jnp.int32)[:, None]
    src4 = (src[None, :] * 2 + off).reshape(NC, NS, NCHUNK, CHUNK)
    dst3 = dst.reshape(NS, NCHUNK, CHUNK)
    b32 = batch.astype(jnp.int32)
    g129 = jnp.arange(NG + 1, dtype=jnp.int32)
    lohi = jnp.sum((b32[:, None] < g129[None, :]).astype(jnp.int32), axis=0)
    lo = lohi[:NG].reshape(1, NG)
    hi = lohi[1:].reshape(1, NG)

    zeros = jnp.zeros((N, CH), jnp.float32)
    sc_degree, sc_conv = _sc_kernels()
    ei32 = edge_index.astype(jnp.int32)
    deg_part = sc_degree(ei32)
    xw1s, dinv = _tc1(x, W1, deg_part)
    p1 = sc_conv(xw1s.reshape(NC * N, CH), zeros, src4, dst3)
    xw2s = _tc2(p1, xw1s, dinv, b1, g1, be1, W2)
    p2 = sc_conv(xw2s.reshape(NC * N, CH), zeros, src4, dst3)
    return _tc3(p2, xw2s, dinv, b2, g2, be2, lo, hi, Wl, bl)


# speedup vs baseline: 1.3647x; 1.0139x over previous
import functools

import jax
import jax.numpy as jnp
from jax import lax
from jax.experimental import pallas as pl
from jax.experimental.pallas import tpu as pltpu
from jax.experimental.pallas import tpu_sc as plsc

N = 10000
E = 320000
C = 128
CH = C // 2
NG = 128

NC = 2
NS = 16
NW = NC * NS
EPW = E // NW
EPS = E // NS
CHUNK = 125
NCHUNK = EPS // CHUNK
SUB_ROWS = 624
TAIL_OFF = NS * SUB_ROWS
TAIL_ROWS = N - TAIL_OFF

DEPTH = 5
LOOK = 4


def _degree_body(ei_hbm, out_hbm, dst_v, deg_v):
    cid = lax.axis_index("c")
    sid = lax.axis_index("s")
    wid = cid * NS + sid
    pltpu.sync_copy(ei_hbm.at[1, pl.ds(wid * EPW, EPW)], dst_v)

    def zero_body(i, _):
        deg_v[pl.ds(i * 16, 16)] = jnp.zeros((16,), jnp.float32)
        return 0

    lax.fori_loop(0, N // 16, zero_body, 0)

    ones = jnp.ones((16,), jnp.float32)

    def add_body(i, _):
        idx = dst_v[pl.ds(i * 16, 16)]
        plsc.addupdate_scatter(deg_v, [idx], ones)
        return 0

    lax.fori_loop(0, EPW // 16, add_body, 0)
    pltpu.sync_copy(deg_v, out_hbm.at[wid])


def _conv_body(ts_hbm, zeros_hbm, src_hbm, dst_hbm, out_hbm,
               src_v, dst_v, *rest):
    bufs = rest[:DEPTH]
    acc_sh = rest[DEPTH]
    gsem = rest[DEPTH + 1:2 * DEPTH + 1]
    ssem = rest[2 * DEPTH + 1:]
    cid = lax.axis_index("c")
    sid = lax.axis_index("s")
    pltpu.sync_copy(src_hbm.at[cid, sid], src_v)
    pltpu.sync_copy(dst_hbm.at[sid], dst_v)

    base = sid * SUB_ROWS
    pltpu.sync_copy(zeros_hbm.at[pl.ds(base, SUB_ROWS)],
                    acc_sh.at[pl.ds(base, SUB_ROWS)])

    @pl.when(sid == NS - 1)
    def _():
        pltpu.sync_copy(zeros_hbm.at[pl.ds(TAIL_OFF, TAIL_ROWS)],
                        acc_sh.at[pl.ds(TAIL_OFF, TAIL_ROWS)])

    plsc.subcore_barrier()

    def gstart(k, j):
        pltpu.async_copy(ts_hbm.at[src_v.at[k]], bufs[j], gsem[j])

    def gdrain(j):
        pltpu.make_async_copy(ts_hbm.at[src_v.at[0]], bufs[j], gsem[j]).wait()

    def sstart(k, j):
        pltpu.async_copy(bufs[j], acc_sh.at[dst_v.at[k]], ssem[j], add=True)

    def sdrain(j):
        pltpu.make_async_copy(bufs[j], acc_sh.at[dst_v.at[0]], ssem[j]).wait()

    for j in range(DEPTH):
        gstart(j, j)
    for j in range(DEPTH - LOOK):
        gdrain(j)
        sstart(j, j)

    def chunk_body(i, _):
        for j in range(DEPTH):
            k = DEPTH * i + j
            sdrain(j)
            gstart(k, j)
            jj = (j + DEPTH - LOOK) % DEPTH
            gdrain(jj)
            sstart(k - LOOK, jj)
        return 0

    lax.fori_loop(1, NCHUNK // DEPTH, chunk_body, 0)
    for t in range(LOOK):
        jj = DEPTH - LOOK + t
        gdrain(jj)
        sstart(NCHUNK - LOOK + t, jj)
    for j in range(DEPTH):
        sdrain(j)
    plsc.subcore_barrier()

    col = cid * CH
    pltpu.sync_copy(acc_sh.at[pl.ds(base, SUB_ROWS)],
                    out_hbm.at[pl.ds(base, SUB_ROWS), pl.ds(col, CH)])

    @pl.when(sid == NS - 1)
    def _():
        pltpu.sync_copy(acc_sh.at[pl.ds(TAIL_OFF, TAIL_ROWS)],
                        out_hbm.at[pl.ds(TAIL_OFF, TAIL_ROWS), pl.ds(col, CH)])


@functools.cache
def _sc_kernels():
    mesh = plsc.VectorSubcoreMesh(core_axis_name="c", subcore_axis_name="s")
    params = pltpu.CompilerParams(needs_layout_passes=False,
                                  use_tc_tiling_on_sc=False)
    sc_degree = functools.partial(
        pl.kernel,
        out_type=jax.ShapeDtypeStruct((NW, N), jnp.float32),
        mesh=mesh,
        compiler_params=params,
        scratch_types=[
            pltpu.VMEM((EPW,), jnp.int32),
            pltpu.VMEM((N,), jnp.float32),
        ],
    )(_degree_body)
    sc_conv = functools.partial(
        pl.kernel,
        out_type=jax.ShapeDtypeStruct((N, C), jnp.float32),
        mesh=mesh,
        compiler_params=params,
        scratch_types=[
            pltpu.VMEM((NCHUNK, CHUNK), jnp.int32),
            pltpu.VMEM((NCHUNK, CHUNK), jnp.int32),
        ] + [pltpu.VMEM((CHUNK, CH), jnp.float32)] * DEPTH
          + [
            pltpu.VMEM_SHARED((N, CH), jnp.float32),
        ] + [pltpu.SemaphoreType.DMA] * (2 * DEPTH),
    )(_conv_body)
    return sc_degree, sc_conv


def _tc1_body(x_ref, w1_ref, part_ref, xw1s_ref, dinv_ref):
    ones = jnp.ones((NW, C), jnp.float32)
    dn = (((0,), (0,)), ((), ()))
    deg = lax.dot_general(part_ref[...], ones, dn,
                          preferred_element_type=jnp.float32) + 1.0
    dinv = lax.rsqrt(deg)
    xw = jnp.dot(x_ref[...], w1_ref[...], preferred_element_type=jnp.float32)
    xw1s_ref[...] = xw * dinv
    dinv_ref[...] = dinv


_tc1 = pl.pallas_call(
    _tc1_body,
    out_shape=(jax.ShapeDtypeStruct((N, C), jnp.float32),
               jax.ShapeDtypeStruct((N, C), jnp.float32)),
)


def _bn_relu(s, g, be):
    mu = jnp.mean(s, axis=0, keepdims=True)
    xc = s - mu
    var = jnp.mean(xc * xc, axis=0, keepdims=True)
    h = xc * lax.rsqrt(var + 1e-5) * g + be
    return jnp.maximum(h, 0.0)


def _tc2_body(p_ref, ts_ref, dinv_ref, b_ref, g_ref, be_ref, w_ref, out_ref):
    dinv = dinv_ref[...]
    s = (p_ref[...] + ts_ref[...]) * dinv + b_ref[...]
    h = _bn_relu(s, g_ref[...], be_ref[...])
    out_ref[...] = jnp.dot(h, w_ref[...],
                           preferred_element_type=jnp.float32) * dinv


_tc2 = pl.pallas_call(
    _tc2_body,
    out_shape=jax.ShapeDtypeStruct((N, C), jnp.float32),
)


def _tc3_body(p_ref, ts_ref, dinv_ref, b_ref, g_ref, be_ref, lo_ref, hi_ref,
              wl_ref, bl_ref, out_ref):
    s = (p_ref[...] + ts_ref[...]) * dinv_ref[...] + b_ref[...]
    h = _bn_relu(s, g_ref[...], be_ref[...])
    lo, hi = lo_ref[...], hi_ref[...]
    ii = lax.broadcasted_iota(jnp.int32, (N, NG), 0)
    m = ((ii >= lo) & (ii < hi)).astype(jnp.float32)
    cnt = (hi - lo).astype(jnp.float32)
    m = m * (1.0 / jnp.maximum(cnt, 1.0))
    dn = (((0,), (0,)), ((), ()))
    pooled = lax.dot_general(m, h, dn, preferred_element_type=jnp.float32)
    out_ref[...] = jnp.dot(pooled, wl_ref[...],
                           preferred_element_type=jnp.float32) + bl_ref[...]


_tc3 = pl.pallas_call(
    _tc3_body,
    out_shape=jax.ShapeDtypeStruct((NG, C), jnp.float32),
)


def kernel(x, edge_index, batch, W1, b1, g1, be1, W2, b2, g2, be2, Wl, bl):
    src = edge_index[0].astype(jnp.int32)
    dst = edge_index[1].astype(jnp.int32)
    off = jnp.arange(NC, dtype=jnp.int32)[:, None]
    src4 = (src[None, :] * 2 + off).reshape(NC, NS, NCHUNK, CHUNK)
    dst3 = dst.reshape(NS, NCHUNK, CHUNK)
    b32 = batch.astype(jnp.int32)
    g129 = jnp.arange(NG + 1, dtype=jnp.int32)
    lohi = jnp.sum((b32[:, None] < g129[None, :]).astype(jnp.int32), axis=0)
    lo = lohi[:NG].reshape(1, NG)
    hi = lohi[1:].reshape(1, NG)

    zeros = jnp.zeros((N, CH), jnp.float32)
    sc_degree, sc_conv = _sc_kernels()
    ei32 = edge_index.astype(jnp.int32)
    deg_part = sc_degree(ei32)
    xw1s, dinv = _tc1(x, W1, deg_part)
    p1 = sc_conv(xw1s.reshape(NC * N, CH), zeros, src4, dst3)
    xw2s = _tc2(p1, xw1s, dinv, b1, g1, be1, W2)
    p2 = sc_conv(xw2s.reshape(NC * N, CH), zeros, src4, dst3)
    return _tc3(p2, xw2s, dinv, b2, g2, be2, lo, hi, Wl, bl)
